# Initial kernel scaffold; baseline (speedup 1.0000x reference)
import jax
import jax.numpy as jnp
from jax.experimental import pallas as pl


def kernel(x, edge_index, W1, b1, W2, b2, fcW, fcb):
    raise NotImplementedError("write your pallas kernel here")


# speedup vs baseline: 82.4214x; 82.4214x over previous
import functools
import jax
import jax.numpy as jnp
from jax import lax
from jax.experimental import pallas as pl
from jax.experimental.pallas import tpu as pltpu
from jax.experimental.pallas import tpu_sc as plsc

NC = 2
NS = 16
NT = NC * NS
CHUNK = 5000

_MESH = dict(core_axis_name="c", subcore_axis_name="s", num_cores=NC,
             num_subcores=NS)


def _round_up(a, m):
    return (a + m - 1) // m * m


def _gs_kernel(npad, e_pad, F, gather):
    per_tile = e_pad // NT
    n_chunks = per_tile // CHUNK
    slice_sz = npad // NS

    scratch = [
        pltpu.VMEM((CHUNK,), jnp.int32),
        pltpu.VMEM((CHUNK,), jnp.int32),
        pltpu.VMEM((CHUNK, F), jnp.float32),
        pltpu.VMEM_SHARED((npad, F), jnp.float32),
        pltpu.SemaphoreType.DMA,
    ]

    @functools.partial(
        pl.kernel,
        out_type=jax.ShapeDtypeStruct((NC, npad, F), jnp.float32),
        mesh=plsc.VectorSubcoreMesh(**_MESH),
        scratch_types=scratch,
        compiler_params=pltpu.CompilerParams(use_tc_tiling_on_sc=False),
    )
    def k(src_hbm, dst_hbm, table_hbm, zeros_hbm, out_hbm,
          sidx, didx, rows, acc_sh, sem):
        c = lax.axis_index("c")
        s = lax.axis_index("s")
        sl = pl.ds(s * slice_sz, slice_sz)
        pltpu.sync_copy(zeros_hbm.at[sl], acc_sh.at[sl])
        if not gather:
            pltpu.sync_copy(table_hbm, rows)
        plsc.subcore_barrier()

        base = (c * NS + s) * per_tile

        def step(kk, _):
            off = base + kk * CHUNK
            pltpu.sync_copy(dst_hbm.at[pl.ds(off, CHUNK)], didx)
            if gather:
                pltpu.sync_copy(src_hbm.at[pl.ds(off, CHUNK)], sidx)
                pltpu.async_copy(table_hbm.at[sidx], rows, sem).wait()
            pltpu.sync_copy(rows, acc_sh.at[didx], add=True)
            return 0

        lax.fori_loop(0, n_chunks, step, 0)
        plsc.subcore_barrier()
        pltpu.sync_copy(acc_sh.at[sl], out_hbm.at[c].at[sl])

    return k


def _prep_call(npad, deg_p, x_pad):
    def body(degp, xp, dis, xn):
        deg = degp[0][:, 0:1] + degp[1][:, 0:1] + 1.0
        d = lax.rsqrt(deg)
        dis[...] = d
        xn[...] = xp[...] * d

    R = 2048
    g = npad // R
    return pl.pallas_call(
        body,
        grid=(g,),
        in_specs=[
            pl.BlockSpec((NC, R, 8), lambda i: (0, i, 0)),
            pl.BlockSpec((R, 8), lambda i: (i, 0)),
        ],
        out_specs=(
            pl.BlockSpec((R, 1), lambda i: (i, 0)),
            pl.BlockSpec((R, 8), lambda i: (i, 0)),
        ),
        out_shape=(
            jax.ShapeDtypeStruct((npad, 1), jnp.float32),
            jax.ShapeDtypeStruct((npad, 8), jnp.float32),
        ),
    )(deg_p, x_pad)


def _mid_call(npad, t1_p, xn, dis, W1, b1, W2):
    R = 2048
    g = npad // R
    assert npad % R == 0

    def body(tp, xnb, disb, w1, bb1, w2, gn):
        agg = (tp[0] + tp[1] + xnb[...]) * disb[...]
        h1 = jnp.dot(agg, w1[...], preferred_element_type=jnp.float32)
        h1 = jnp.maximum(h1 + bb1[...], 0.0)
        gn[...] = jnp.dot(h1, w2[...],
                          preferred_element_type=jnp.float32) * disb[...]

    return pl.pallas_call(
        body,
        grid=(g,),
        in_specs=[
            pl.BlockSpec((NC, R, 8), lambda i: (0, i, 0)),
            pl.BlockSpec((R, 8), lambda i: (i, 0)),
            pl.BlockSpec((R, 1), lambda i: (i, 0)),
            pl.BlockSpec((8, 16), lambda i: (0, 0)),
            pl.BlockSpec((1, 16), lambda i: (0, 0)),
            pl.BlockSpec((16, 8), lambda i: (0, 0)),
        ],
        out_specs=pl.BlockSpec((R, 8), lambda i: (i, 0)),
        out_shape=jax.ShapeDtypeStruct((npad, 8), jnp.float32),
    )(t1_p, xn, dis, W1, b1.reshape(1, 16), W2)


def _final_call(n, t2_p, gn, dis, fcW, b2, fcb):
    R = 4000
    g = n // R
    assert n % R == 0

    def body(tp, gnb, disb, w, bb2, bfc, out):
        agg = (tp[0] + tp[1] + gnb[...]) * disb[...]
        cvec = jnp.dot(bb2[...], w[...],
                       preferred_element_type=jnp.float32) + bfc[...]
        out[...] = jnp.dot(agg, w[...],
                           preferred_element_type=jnp.float32) + cvec

    return pl.pallas_call(
        body,
        grid=(g,),
        in_specs=[
            pl.BlockSpec((NC, R, 8), lambda i: (0, i, 0)),
            pl.BlockSpec((R, 8), lambda i: (i, 0)),
            pl.BlockSpec((R, 1), lambda i: (i, 0)),
            pl.BlockSpec((8, 256), lambda i: (0, 0)),
            pl.BlockSpec((1, 8), lambda i: (0, 0)),
            pl.BlockSpec((1, 256), lambda i: (0, 0)),
        ],
        out_specs=pl.BlockSpec((R, 256), lambda i: (i, 0)),
        out_shape=jax.ShapeDtypeStruct((n, 256), jnp.float32),
    )(t2_p, gn, dis, fcW, b2.reshape(1, 8), fcb.reshape(1, 256))


@jax.jit
def kernel(x, edge_index, W1, b1, W2, b2, fcW, fcb):
    n = x.shape[0]
    e = edge_index.shape[1]
    npad = _round_up(n + 1, 2048)
    e_pad = _round_up(e, NT * CHUNK)

    src = edge_index[0]
    dst = edge_index[1]
    if e_pad != e:
        pad = jnp.full((e_pad - e,), n, dtype=jnp.int32)
        src = jnp.concatenate([src, pad])
        dst = jnp.concatenate([dst, pad])

    x_pad = jnp.zeros((npad, 8), jnp.float32).at[:n, :2].set(x)
    W1p = jnp.zeros((8, 16), jnp.float32).at[:2].set(W1)
    zeros8 = jnp.zeros((npad, 8), jnp.float32)
    ones_c = jnp.ones((CHUNK, 8), jnp.float32)

    deg_p = _gs_kernel(npad, e_pad, 8, gather=False)(src, dst, ones_c, zeros8)

    dis, xn = _prep_call(npad, deg_p, x_pad)

    t1_p = _gs_kernel(npad, e_pad, 8, gather=True)(src, dst, xn, zeros8)

    gn = _mid_call(npad, t1_p, xn, dis, W1p, b1, W2)

    t2_p = _gs_kernel(npad, e_pad, 8, gather=True)(src, dst, gn, zeros8)

    return _final_call(n, t2_p, gn, dis, fcW, b2, fcb)


# speedup vs baseline: 86.8567x; 1.0538x over previous
import functools
import jax
import jax.numpy as jnp
from jax import lax
from jax.experimental import pallas as pl
from jax.experimental.pallas import tpu as pltpu
from jax.experimental.pallas import tpu_sc as plsc

NC = 2
NS = 16
NT = NC * NS
CHUNK = 4000

_MESH = dict(core_axis_name="c", subcore_axis_name="s", num_cores=NC,
             num_subcores=NS)


def _round_up(a, m):
    return (a + m - 1) // m * m


def _gs_kernel(npad, e_pad, F, gather):
    per_tile = e_pad // NT
    n_chunks = per_tile // CHUNK
    slice_sz = npad // NS

    scratch = [
        pltpu.VMEM((CHUNK,), jnp.int32),
        pltpu.VMEM((CHUNK,), jnp.int32),
        pltpu.VMEM((CHUNK,), jnp.int32),
        pltpu.VMEM((CHUNK,), jnp.int32),
        pltpu.VMEM((CHUNK, F), jnp.float32),
        pltpu.VMEM((CHUNK, F), jnp.float32),
        pltpu.VMEM_SHARED((npad, F), jnp.float32),
        pltpu.SemaphoreType.DMA,
        pltpu.SemaphoreType.DMA,
        pltpu.SemaphoreType.DMA,
        pltpu.SemaphoreType.DMA,
        pltpu.SemaphoreType.DMA,
        pltpu.SemaphoreType.DMA,
    ]

    @functools.partial(
        pl.kernel,
        out_type=jax.ShapeDtypeStruct((NC, npad, F), jnp.float32),
        mesh=plsc.VectorSubcoreMesh(**_MESH),
        scratch_types=scratch,
        compiler_params=pltpu.CompilerParams(use_tc_tiling_on_sc=False),
    )
    def k(src_hbm, dst_hbm, table_hbm, zeros_hbm, out_hbm,
          sidx0, sidx1, didx0, didx1, rows0, rows1, acc_sh,
          isem0, isem1, gsem0, gsem1, ssem0, ssem1):
        sidx = (sidx0, sidx1)
        didx = (didx0, didx1)
        rows = (rows0, rows1)
        isem = (isem0, isem1)
        gsem = (gsem0, gsem1)
        ssem = (ssem0, ssem1)
        c = lax.axis_index("c")
        s = lax.axis_index("s")
        sl = pl.ds(s * slice_sz, slice_sz)
        pltpu.sync_copy(zeros_hbm.at[sl], acc_sh.at[sl])
        if not gather:
            pltpu.sync_copy(table_hbm, rows[0])
            pltpu.sync_copy(table_hbm, rows[1])
        plsc.subcore_barrier()

        base = (c * NS + s) * per_tile

        def off(kk):
            return pl.ds(base + kk * CHUNK, CHUNK)

        def idx_start(kk, b):
            pltpu.async_copy(dst_hbm.at[off(kk)], didx[b], isem[b])
            if gather:
                pltpu.async_copy(src_hbm.at[off(kk)], sidx[b], isem[b])

        def idx_wait(kk, b):
            pltpu.make_async_copy(dst_hbm.at[off(kk)], didx[b], isem[b]).wait()
            if gather:
                pltpu.make_async_copy(src_hbm.at[off(kk)], sidx[b],
                                      isem[b]).wait()

        def gather_start(b):
            if gather:
                pltpu.async_copy(table_hbm.at[sidx[b]], rows[b], gsem[b])

        def gather_wait(b):
            if gather:
                pltpu.make_async_copy(table_hbm.at[sidx[b]], rows[b],
                                      gsem[b]).wait()

        idx_start(0, 0)
        idx_wait(0, 0)
        gather_start(0)
        for kk in range(n_chunks):
            b = kk & 1
            nb = 1 - b
            gather_wait(b)
            if kk + 1 < n_chunks:
                idx_start(kk + 1, nb)
                idx_wait(kk + 1, nb)
                gather_start(nb)
            pltpu.sync_copy(rows[b], acc_sh.at[didx[b]], add=True)

        plsc.subcore_barrier()
        pltpu.sync_copy(acc_sh.at[sl], out_hbm.at[c].at[sl])

    return k


def _prep_call(npad, deg_p, x_pad):
    def body(degp, xp, dis, xn):
        deg = degp[0][:, 0:1] + degp[1][:, 0:1] + 1.0
        d = lax.rsqrt(deg)
        dis[...] = d
        xn[...] = xp[...] * d

    R = 2048
    g = npad // R
    return pl.pallas_call(
        body,
        grid=(g,),
        in_specs=[
            pl.BlockSpec((NC, R, 8), lambda i: (0, i, 0)),
            pl.BlockSpec((R, 8), lambda i: (i, 0)),
        ],
        out_specs=(
            pl.BlockSpec((R, 1), lambda i: (i, 0)),
            pl.BlockSpec((R, 8), lambda i: (i, 0)),
        ),
        out_shape=(
            jax.ShapeDtypeStruct((npad, 1), jnp.float32),
            jax.ShapeDtypeStruct((npad, 8), jnp.float32),
        ),
    )(deg_p, x_pad)


def _mid_call(npad, t1_p, xn, dis, W1, b1, W2):
    R = 2048
    g = npad // R
    assert npad % R == 0

    def body(tp, xnb, disb, w1, bb1, w2, gn):
        agg = (tp[0] + tp[1] + xnb[...]) * disb[...]
        h1 = jnp.dot(agg, w1[...], preferred_element_type=jnp.float32)
        h1 = jnp.maximum(h1 + bb1[...], 0.0)
        gn[...] = jnp.dot(h1, w2[...],
                          preferred_element_type=jnp.float32) * disb[...]

    return pl.pallas_call(
        body,
        grid=(g,),
        in_specs=[
            pl.BlockSpec((NC, R, 8), lambda i: (0, i, 0)),
            pl.BlockSpec((R, 8), lambda i: (i, 0)),
            pl.BlockSpec((R, 1), lambda i: (i, 0)),
            pl.BlockSpec((8, 16), lambda i: (0, 0)),
            pl.BlockSpec((1, 16), lambda i: (0, 0)),
            pl.BlockSpec((16, 8), lambda i: (0, 0)),
        ],
        out_specs=pl.BlockSpec((R, 8), lambda i: (i, 0)),
        out_shape=jax.ShapeDtypeStruct((npad, 8), jnp.float32),
    )(t1_p, xn, dis, W1, b1.reshape(1, 16), W2)


def _final_call(n, t2_p, gn, dis, fcW, b2, fcb):
    R = 4000
    g = n // R
    assert n % R == 0

    def body(tp, gnb, disb, w, bb2, bfc, out):
        agg = (tp[0] + tp[1] + gnb[...]) * disb[...]
        cvec = jnp.dot(bb2[...], w[...],
                       preferred_element_type=jnp.float32) + bfc[...]
        out[...] = jnp.dot(agg, w[...],
                           preferred_element_type=jnp.float32) + cvec

    return pl.pallas_call(
        body,
        grid=(g,),
        in_specs=[
            pl.BlockSpec((NC, R, 8), lambda i: (0, i, 0)),
            pl.BlockSpec((R, 8), lambda i: (i, 0)),
            pl.BlockSpec((R, 1), lambda i: (i, 0)),
            pl.BlockSpec((8, 256), lambda i: (0, 0)),
            pl.BlockSpec((1, 8), lambda i: (0, 0)),
            pl.BlockSpec((1, 256), lambda i: (0, 0)),
        ],
        out_specs=pl.BlockSpec((R, 256), lambda i: (i, 0)),
        out_shape=jax.ShapeDtypeStruct((n, 256), jnp.float32),
    )(t2_p, gn, dis, fcW, b2.reshape(1, 8), fcb.reshape(1, 256))


@jax.jit
def kernel(x, edge_index, W1, b1, W2, b2, fcW, fcb):
    n = x.shape[0]
    e = edge_index.shape[1]
    npad = _round_up(n + 1, 2048)
    e_pad = _round_up(e, NT * CHUNK)

    src = edge_index[0]
    dst = edge_index[1]
    if e_pad != e:
        pad = jnp.full((e_pad - e,), n, dtype=jnp.int32)
        src = jnp.concatenate([src, pad])
        dst = jnp.concatenate([dst, pad])

    x_pad = jnp.zeros((npad, 8), jnp.float32).at[:n, :2].set(x)
    W1p = jnp.zeros((8, 16), jnp.float32).at[:2].set(W1)
    zeros8 = jnp.zeros((npad, 8), jnp.float32)
    ones_c = jnp.ones((CHUNK, 8), jnp.float32)

    deg_p = _gs_kernel(npad, e_pad, 8, gather=False)(src, dst, ones_c, zeros8)

    dis, xn = _prep_call(npad, deg_p, x_pad)

    t1_p = _gs_kernel(npad, e_pad, 8, gather=True)(src, dst, xn, zeros8)

    gn = _mid_call(npad, t1_p, xn, dis, W1p, b1, W2)

    t2_p = _gs_kernel(npad, e_pad, 8, gather=True)(src, dst, gn, zeros8)

    return _final_call(n, t2_p, gn, dis, fcW, b2, fcb)


# speedup vs baseline: 146.4079x; 1.6856x over previous
import functools
import jax
import jax.numpy as jnp
from jax import lax
from jax.experimental import pallas as pl
from jax.experimental.pallas import tpu as pltpu
from jax.experimental.pallas import tpu_sc as plsc

NC = 2
NS = 16
NT = NC * NS
CHUNK = 4000

_MESH = dict(core_axis_name="c", subcore_axis_name="s", num_cores=NC,
             num_subcores=NS)


def _round_up(a, m):
    return (a + m - 1) // m * m


def _gs_kernel(npad, e_pad, F, gather):
    per_tile = e_pad // NT
    n_chunks = per_tile // CHUNK
    slice_sz = npad // NS

    scratch = [
        pltpu.VMEM((CHUNK,), jnp.int32),
        pltpu.VMEM((CHUNK,), jnp.int32),
        pltpu.VMEM((CHUNK,), jnp.int32),
        pltpu.VMEM((CHUNK,), jnp.int32),
        pltpu.VMEM((CHUNK, F), jnp.float32),
        pltpu.VMEM((CHUNK, F), jnp.float32),
        pltpu.VMEM_SHARED((npad, F), jnp.float32),
        pltpu.SemaphoreType.DMA,
        pltpu.SemaphoreType.DMA,
        pltpu.SemaphoreType.DMA,
        pltpu.SemaphoreType.DMA,
        pltpu.SemaphoreType.DMA,
        pltpu.SemaphoreType.DMA,
    ]

    @functools.partial(
        pl.kernel,
        out_type=jax.ShapeDtypeStruct((NC, npad, F), jnp.float32),
        mesh=plsc.VectorSubcoreMesh(**_MESH),
        scratch_types=scratch,
        compiler_params=pltpu.CompilerParams(use_tc_tiling_on_sc=False),
    )
    def k(src_hbm, dst_hbm, table_hbm, zeros_hbm, out_hbm,
          sidx0, sidx1, didx0, didx1, rows0, rows1, acc_sh,
          isem0, isem1, gsem0, gsem1, ssem0, ssem1):
        sidx = (sidx0, sidx1)
        didx = (didx0, didx1)
        rows = (rows0, rows1)
        isem = (isem0, isem1)
        gsem = (gsem0, gsem1)
        ssem = (ssem0, ssem1)
        c = lax.axis_index("c")
        s = lax.axis_index("s")
        sl = pl.ds(s * slice_sz, slice_sz)
        pltpu.sync_copy(zeros_hbm.at[sl], acc_sh.at[sl])
        if not gather:
            pltpu.sync_copy(table_hbm, rows[0])
            pltpu.sync_copy(table_hbm, rows[1])
        plsc.subcore_barrier()

        base = (c * NS + s) * per_tile

        def off(kk):
            return pl.ds(base + kk * CHUNK, CHUNK)

        def idx_start(kk, b):
            pltpu.async_copy(dst_hbm.at[off(kk)], didx[b], isem[b])
            if gather:
                pltpu.async_copy(src_hbm.at[off(kk)], sidx[b], isem[b])

        def idx_wait(kk, b):
            pltpu.make_async_copy(dst_hbm.at[off(kk)], didx[b], isem[b]).wait()
            if gather:
                pltpu.make_async_copy(src_hbm.at[off(kk)], sidx[b],
                                      isem[b]).wait()

        def gather_start(b):
            if gather:
                pltpu.async_copy(table_hbm.at[sidx[b]], rows[b], gsem[b])

        def gather_wait(b):
            if gather:
                pltpu.make_async_copy(table_hbm.at[sidx[b]], rows[b],
                                      gsem[b]).wait()

        idx_start(0, 0)
        idx_wait(0, 0)
        gather_start(0)
        for kk in range(n_chunks):
            b = kk & 1
            nb = 1 - b
            gather_wait(b)
            if kk + 1 < n_chunks:
                idx_start(kk + 1, nb)
                idx_wait(kk + 1, nb)
                gather_start(nb)
            pltpu.sync_copy(rows[b], acc_sh.at[didx[b]], add=True)

        plsc.subcore_barrier()
        pltpu.sync_copy(acc_sh.at[sl], out_hbm.at[c].at[sl])

    return k


def _prep_call(npad, deg_p, x_flat):
    def body(degp, xf, disf, xnf):
        deg = degp[0] + degp[1] + 1.0
        d = lax.rsqrt(deg)
        disf[...] = d
        xnf[...] = xf[...] * d

    G = npad // 16
    R = 784
    g = G // R
    return pl.pallas_call(
        body,
        grid=(g,),
        in_specs=[
            pl.BlockSpec((NC, R, 128), lambda i: (0, i, 0)),
            pl.BlockSpec((R, 128), lambda i: (i, 0)),
        ],
        out_specs=(
            pl.BlockSpec((R, 128), lambda i: (i, 0)),
            pl.BlockSpec((R, 128), lambda i: (i, 0)),
        ),
        out_shape=(
            jax.ShapeDtypeStruct((G, 128), jnp.float32),
            jax.ShapeDtypeStruct((G, 128), jnp.float32),
        ),
    )(deg_p, x_flat)


def _mid_call(npad, t1_p, xnf, disf, W1big, b1big, W2big):
    G = npad // 16
    R = 784
    g = G // R

    def body(tp, xnb, disb, w1, bb1, w2, gnf):
        agg = (tp[0] + tp[1] + xnb[...]) * disb[...]
        h1 = jnp.dot(agg, w1[...], preferred_element_type=jnp.float32)
        h1 = jnp.maximum(h1 + bb1[...], 0.0)
        gnf[...] = jnp.dot(h1, w2[...],
                           preferred_element_type=jnp.float32) * disb[...]

    return pl.pallas_call(
        body,
        grid=(g,),
        in_specs=[
            pl.BlockSpec((NC, R, 128), lambda i: (0, i, 0)),
            pl.BlockSpec((R, 128), lambda i: (i, 0)),
            pl.BlockSpec((R, 128), lambda i: (i, 0)),
            pl.BlockSpec((128, 256), lambda i: (0, 0)),
            pl.BlockSpec((1, 256), lambda i: (0, 0)),
            pl.BlockSpec((256, 128), lambda i: (0, 0)),
        ],
        out_specs=pl.BlockSpec((R, 128), lambda i: (i, 0)),
        out_shape=jax.ShapeDtypeStruct((G, 128), jnp.float32),
    )(t1_p, xnf, disf, W1big, b1big.reshape(1, 256), W2big)


def _post_call(npad, t2_p, gnf, disf):
    G = npad // 16
    R = 784
    g = G // R

    def body(tp, gnb, disb, aggf):
        aggf[...] = (tp[0] + tp[1] + gnb[...]) * disb[...]

    return pl.pallas_call(
        body,
        grid=(g,),
        in_specs=[
            pl.BlockSpec((NC, R, 128), lambda i: (0, i, 0)),
            pl.BlockSpec((R, 128), lambda i: (i, 0)),
            pl.BlockSpec((R, 128), lambda i: (i, 0)),
        ],
        out_specs=pl.BlockSpec((R, 128), lambda i: (i, 0)),
        out_shape=jax.ShapeDtypeStruct((G, 128), jnp.float32),
    )(t2_p, gnf, disf)


def _final_call(n, agg2, fcW, b2, fcb):
    R = 4000
    g = n // R

    def body(ab, w, bb2, bfc, out):
        cvec = jnp.dot(bb2[...], w[...],
                       preferred_element_type=jnp.float32) + bfc[...]
        out[...] = jnp.dot(ab[...], w[...],
                           preferred_element_type=jnp.float32) + cvec

    return pl.pallas_call(
        body,
        grid=(g,),
        in_specs=[
            pl.BlockSpec((R, 8), lambda i: (i, 0)),
            pl.BlockSpec((8, 256), lambda i: (0, 0)),
            pl.BlockSpec((1, 8), lambda i: (0, 0)),
            pl.BlockSpec((1, 256), lambda i: (0, 0)),
        ],
        out_specs=pl.BlockSpec((R, 256), lambda i: (i, 0)),
        out_shape=jax.ShapeDtypeStruct((n, 256), jnp.float32),
    )(agg2, fcW, b2.reshape(1, 8), fcb.reshape(1, 256))


@jax.jit
def kernel(x, edge_index, W1, b1, W2, b2, fcW, fcb):
    n = x.shape[0]
    e = edge_index.shape[1]
    npad = _round_up(n + 1, 2048)
    e_pad = _round_up(e, NT * CHUNK)

    src = edge_index[0]
    dst = edge_index[1]
    if e_pad != e:
        pad = jnp.full((e_pad - e,), n, dtype=jnp.int32)
        src = jnp.concatenate([src, pad])
        dst = jnp.concatenate([dst, pad])

    G = npad // 16
    x_flat = jnp.zeros((npad, 8), jnp.float32).at[:n, :2].set(x)
    x_flat = x_flat.reshape(G, 128)
    W1p = jnp.zeros((8, 16), jnp.float32).at[:2].set(W1)
    W1big = jnp.kron(jnp.eye(16, dtype=jnp.float32), W1p)
    W2big = jnp.kron(jnp.eye(16, dtype=jnp.float32), W2)
    b1big = jnp.tile(b1, 16)
    zeros8 = jnp.zeros((npad, 8), jnp.float32)
    ones_c = jnp.ones((CHUNK, 8), jnp.float32)

    deg_p = _gs_kernel(npad, e_pad, 8, gather=False)(src, dst, ones_c, zeros8)

    disf, xnf = _prep_call(npad, deg_p.reshape(NC, G, 128), x_flat)

    t1_p = _gs_kernel(npad, e_pad, 8, gather=True)(
        src, dst, xnf.reshape(npad, 8), zeros8)

    gnf = _mid_call(npad, t1_p.reshape(NC, G, 128), xnf, disf,
                    W1big, b1big, W2big)

    t2_p = _gs_kernel(npad, e_pad, 8, gather=True)(
        src, dst, gnf.reshape(npad, 8), zeros8)

    agg2f = _post_call(npad, t2_p.reshape(NC, G, 128), gnf, disf)
    return _final_call(n, agg2f.reshape(npad, 8), fcW, b2, fcb)
